# Initial kernel scaffold; baseline (speedup 1.0000x reference)
import jax
import jax.numpy as jnp
from jax.experimental import pallas as pl


def kernel(Z):
    raise NotImplementedError("write your pallas kernel here")


# speedup vs baseline: 8.3680x; 8.3680x over previous
import jax
import jax.numpy as jnp
from jax.experimental import pallas as pl

_BLOCK = 512


def _eq_kernel(zrow_ref, zcol_ref, out_ref):
    zr = zrow_ref[0, :]
    zc = zcol_ref[0, :]
    out_ref[...] = (zr[:, None] == zc[None, :]).astype(jnp.float32)


def kernel(Z):
    n = Z.shape[0]
    z2 = Z.reshape(1, n).astype(jnp.int32)
    grid = n // _BLOCK
    return pl.pallas_call(
        _eq_kernel,
        grid=(grid,),
        in_specs=[
            pl.BlockSpec((1, _BLOCK), lambda i: (0, i)),
            pl.BlockSpec((1, n), lambda i: (0, 0)),
        ],
        out_specs=pl.BlockSpec((_BLOCK, n), lambda i: (i, 0)),
        out_shape=jax.ShapeDtypeStruct((n, n), jnp.float32),
    )(z2, z2)
